# Initial kernel scaffold; baseline (speedup 1.0000x reference)
import jax
import jax.numpy as jnp
from jax.experimental import pallas as pl


def kernel(logits, labels):
    raise NotImplementedError("write your pallas kernel here")


# speedup vs baseline: 266.5968x; 266.5968x over previous
import functools

import jax
import jax.numpy as jnp
from jax.experimental import pallas as pl
from jax.experimental.pallas import tpu as pltpu


def _loss_body(lref, oref, nacc, sacc, carry, *, nc, rows, w, h, nchunk):
    ncls = nc - 1
    pack = 2048
    b = pl.program_id(0)
    j = pl.program_id(1)

    x = lref[0]

    best = x[0]
    idx = jnp.zeros((rows, w), jnp.int32)
    for c in range(1, nc):
        v = x[c]
        m = v > best
        best = jnp.where(m, v, best)
        idx = jnp.where(m, c, idx)

    col = jax.lax.broadcasted_iota(jnp.int32, (rows, w), 1)
    ns_l, mn_l, mx_l = [], [], []
    for c in range(1, nc):
        m = idx == c
        ns_l.append(jnp.sum(m.astype(jnp.int32), axis=1, keepdims=True))
        mn_l.append(jnp.min(jnp.where(m, col, 2 * w), axis=1, keepdims=True))
        mx_l.append(jnp.max(jnp.where(m, col, -1), axis=1, keepdims=True))
    ns = jnp.concatenate(ns_l, axis=1)
    mn = jnp.concatenate(mn_l, axis=1)
    mx = jnp.concatenate(mx_l, axis=1)

    grow = j * rows + jax.lax.broadcasted_iota(jnp.int32, (rows, ncls), 0)
    occ = ns > 0
    first = mn - grow
    last = mx - grow
    srow = jnp.where(occ, mx - mn, 0)
    packed = jnp.where(occ, (grow + 1) * pack + (last + h), -1)

    @pl.when(j == 0)
    def _():
        nacc[:1, :ncls] = jnp.zeros((1, ncls), jnp.int32)
        sacc[:1, :ncls] = jnp.zeros((1, ncls), jnp.int32)
        carry[:1, :ncls] = jnp.full((1, ncls), -1, jnp.int32)

    e = jnp.concatenate([carry[:1, :ncls], packed[:-1]], axis=0)
    k = 1
    while k < rows:
        pad = jnp.full((k, ncls), -1, jnp.int32)
        e = jnp.maximum(e, jnp.concatenate([pad, e[:-k]], axis=0))
        k *= 2
    prev_ok = e >= 0
    prev_last = (e & (pack - 1)) - h
    cross = jnp.where(occ & prev_ok, jnp.abs(first - prev_last), 0)

    nacc[:1, :ncls] = nacc[:1, :ncls] + jnp.sum(ns, axis=0, keepdims=True)
    sacc[:1, :ncls] = sacc[:1, :ncls] + jnp.sum(srow + cross, axis=0,
                                                keepdims=True)
    carry[:1, :ncls] = jnp.maximum(e[-1:, :], packed[-1:, :])

    @pl.when(jnp.logical_and(b == 0, j == 0))
    def _():
        oref[:, :] = jnp.zeros((1, 1), jnp.float32)

    @pl.when(j == nchunk - 1)
    def _():
        nf = nacc[:1, :ncls].astype(jnp.float32)
        sf = sacc[:1, :ncls].astype(jnp.float32)
        mean = sf / jnp.maximum(nf - 1.0, 1.0)
        contrib = jnp.where(nf >= 2.0, mean / (nf + 1.0), 0.0)
        oref[:, :] = oref[:, :] + jnp.sum(contrib, axis=1, keepdims=True)


def kernel(logits, labels):
    del labels
    bs, nc, h, w = logits.shape
    rows = 64
    nchunk = h // rows

    body = functools.partial(_loss_body, nc=nc, rows=rows, w=w, h=h,
                             nchunk=nchunk)
    out = pl.pallas_call(
        body,
        grid=(bs, nchunk),
        in_specs=[
            pl.BlockSpec((1, nc, rows, w), lambda b, j: (b, 0, j, 0)),
        ],
        out_specs=pl.BlockSpec((1, 1), lambda b, j: (0, 0)),
        out_shape=jax.ShapeDtypeStruct((1, 1), jnp.float32),
        scratch_shapes=[
            pltpu.VMEM((8, 128), jnp.int32),
            pltpu.VMEM((8, 128), jnp.int32),
            pltpu.VMEM((8, 128), jnp.int32),
        ],
        compiler_params=pltpu.CompilerParams(
            dimension_semantics=("arbitrary", "arbitrary"),
        ),
    )(logits)
    return out[0, 0]


# speedup vs baseline: 342.4314x; 1.2845x over previous
import functools

import jax
import jax.numpy as jnp
from jax.experimental import pallas as pl
from jax.experimental.pallas import tpu as pltpu


def _loss_body(lref, oref, nacc, sacc, carry, *, nc, rows, w, h, nchunk):
    ncls = nc - 1
    pack = 2048
    b = pl.program_id(0)
    j = pl.program_id(1)

    x = lref[0]

    best = x[0]
    idx = jnp.zeros((rows, w), jnp.int32)
    for c in range(1, nc):
        v = x[c]
        m = v > best
        best = jnp.where(m, v, best)
        idx = jnp.where(m, c, idx)

    colp = jax.lax.broadcasted_iota(
        jnp.int32, (rows, w), 1).astype(jnp.float32) + 1.0
    colr = jnp.float32(w + 1) - colp
    ns_l, mn_l, mx_l = [], [], []
    for c in range(1, nc):
        mf = jnp.where(idx == c, 1.0, 0.0)
        ns_l.append(jnp.sum(mf, axis=1, keepdims=True))
        mx_l.append(jnp.max(mf * colp, axis=1, keepdims=True))
        mn_l.append(jnp.max(mf * colr, axis=1, keepdims=True))
    ns = jnp.concatenate(ns_l, axis=1).astype(jnp.int32)
    mn = w - jnp.concatenate(mn_l, axis=1).astype(jnp.int32)
    mx = jnp.concatenate(mx_l, axis=1).astype(jnp.int32) - 1

    grow = j * rows + jax.lax.broadcasted_iota(jnp.int32, (rows, ncls), 0)
    occ = ns > 0
    first = mn - grow
    last = mx - grow
    srow = jnp.where(occ, mx - mn, 0)
    packed = jnp.where(occ, (grow + 1) * pack + (last + h), -1)

    @pl.when(j == 0)
    def _():
        nacc[:1, :ncls] = jnp.zeros((1, ncls), jnp.int32)
        sacc[:1, :ncls] = jnp.zeros((1, ncls), jnp.int32)
        carry[:1, :ncls] = jnp.full((1, ncls), -1, jnp.int32)

    e = jnp.concatenate([carry[:1, :ncls], packed[:-1]], axis=0)
    k = 1
    while k < rows:
        pad = jnp.full((k, ncls), -1, jnp.int32)
        e = jnp.maximum(e, jnp.concatenate([pad, e[:-k]], axis=0))
        k *= 2
    prev_ok = e >= 0
    prev_last = (e & (pack - 1)) - h
    cross = jnp.where(occ & prev_ok, jnp.abs(first - prev_last), 0)

    nacc[:1, :ncls] = nacc[:1, :ncls] + jnp.sum(ns, axis=0, keepdims=True)
    sacc[:1, :ncls] = sacc[:1, :ncls] + jnp.sum(srow + cross, axis=0,
                                                keepdims=True)
    carry[:1, :ncls] = jnp.maximum(e[-1:, :], packed[-1:, :])

    @pl.when(jnp.logical_and(b == 0, j == 0))
    def _():
        oref[:, :] = jnp.zeros((1, 1), jnp.float32)

    @pl.when(j == nchunk - 1)
    def _():
        nf = nacc[:1, :ncls].astype(jnp.float32)
        sf = sacc[:1, :ncls].astype(jnp.float32)
        mean = sf / jnp.maximum(nf - 1.0, 1.0)
        contrib = jnp.where(nf >= 2.0, mean / (nf + 1.0), 0.0)
        oref[:, :] = oref[:, :] + jnp.sum(contrib, axis=1, keepdims=True)


def kernel(logits, labels):
    del labels
    bs, nc, h, w = logits.shape
    rows = 64
    nchunk = h // rows

    body = functools.partial(_loss_body, nc=nc, rows=rows, w=w, h=h,
                             nchunk=nchunk)
    out = pl.pallas_call(
        body,
        grid=(bs, nchunk),
        in_specs=[
            pl.BlockSpec((1, nc, rows, w), lambda b, j: (b, 0, j, 0)),
        ],
        out_specs=pl.BlockSpec((1, 1), lambda b, j: (0, 0)),
        out_shape=jax.ShapeDtypeStruct((1, 1), jnp.float32),
        scratch_shapes=[
            pltpu.VMEM((8, 128), jnp.int32),
            pltpu.VMEM((8, 128), jnp.int32),
            pltpu.VMEM((8, 128), jnp.int32),
        ],
        compiler_params=pltpu.CompilerParams(
            dimension_semantics=("arbitrary", "arbitrary"),
        ),
    )(logits)
    return out[0, 0]


# speedup vs baseline: 405.9229x; 1.1854x over previous
import functools

import jax
import jax.numpy as jnp
from jax.experimental import pallas as pl
from jax.experimental.pallas import tpu as pltpu


def _loss_body(lref, oref, nacc, sacc, carry, *, nc, rows, w, h, nchunk):
    ncls = nc - 1
    pack = 2048
    b = pl.program_id(0)
    j = pl.program_id(1)

    x = lref[0]

    best = x[0]
    idx = jnp.zeros((rows, w), jnp.int32)
    for c in range(1, nc):
        v = x[c]
        m = v > best
        best = jnp.where(m, v, best)
        idx = jnp.where(m, c, idx)

    colp = jax.lax.broadcasted_iota(
        jnp.int32, (rows, w), 1).astype(jnp.float32) + 1.0
    colr = jnp.float32(w + 1) - colp
    ns_l, mn_l, mx_l = [], [], []
    for c in range(1, nc):
        mf = jnp.where(idx == c, 1.0, 0.0)
        ns_l.append(jnp.sum(mf, axis=1, keepdims=True))
        mx_l.append(jnp.max(mf * colp, axis=1, keepdims=True))
        mn_l.append(jnp.max(mf * colr, axis=1, keepdims=True))
    ns = jnp.concatenate(ns_l, axis=1).astype(jnp.int32)
    mn = w - jnp.concatenate(mn_l, axis=1).astype(jnp.int32)
    mx = jnp.concatenate(mx_l, axis=1).astype(jnp.int32) - 1

    grow = j * rows + jax.lax.broadcasted_iota(jnp.int32, (rows, ncls), 0)
    occ = ns > 0
    first = mn - grow
    last = mx - grow
    srow = jnp.where(occ, mx - mn, 0)
    packed = jnp.where(occ, (grow + 1) * pack + (last + h), -1)

    @pl.when(j == 0)
    def _():
        nacc[:1, :ncls] = jnp.zeros((1, ncls), jnp.int32)
        sacc[:1, :ncls] = jnp.zeros((1, ncls), jnp.int32)
        carry[:1, :ncls] = jnp.full((1, ncls), -1, jnp.int32)

    e = jnp.concatenate([carry[:1, :ncls], packed[:-1]], axis=0)
    k = 1
    while k < rows:
        pad = jnp.full((k, ncls), -1, jnp.int32)
        e = jnp.maximum(e, jnp.concatenate([pad, e[:-k]], axis=0))
        k *= 2
    prev_ok = e >= 0
    prev_last = (e & (pack - 1)) - h
    cross = jnp.where(occ & prev_ok, jnp.abs(first - prev_last), 0)

    nacc[:1, :ncls] = nacc[:1, :ncls] + jnp.sum(ns, axis=0, keepdims=True)
    sacc[:1, :ncls] = sacc[:1, :ncls] + jnp.sum(srow + cross, axis=0,
                                                keepdims=True)
    carry[:1, :ncls] = jnp.maximum(e[-1:, :], packed[-1:, :])

    @pl.when(jnp.logical_and(b == 0, j == 0))
    def _():
        oref[:, :] = jnp.zeros((1, 1), jnp.float32)

    @pl.when(j == nchunk - 1)
    def _():
        nf = nacc[:1, :ncls].astype(jnp.float32)
        sf = sacc[:1, :ncls].astype(jnp.float32)
        mean = sf / jnp.maximum(nf - 1.0, 1.0)
        contrib = jnp.where(nf >= 2.0, mean / (nf + 1.0), 0.0)
        oref[:, :] = oref[:, :] + jnp.sum(contrib, axis=1, keepdims=True)


def kernel(logits, labels):
    del labels
    bs, nc, h, w = logits.shape
    rows = 128
    nchunk = h // rows

    body = functools.partial(_loss_body, nc=nc, rows=rows, w=w, h=h,
                             nchunk=nchunk)
    out = pl.pallas_call(
        body,
        grid=(bs, nchunk),
        in_specs=[
            pl.BlockSpec((1, nc, rows, w), lambda b, j: (b, 0, j, 0)),
        ],
        out_specs=pl.BlockSpec((1, 1), lambda b, j: (0, 0)),
        out_shape=jax.ShapeDtypeStruct((1, 1), jnp.float32),
        scratch_shapes=[
            pltpu.VMEM((8, 128), jnp.int32),
            pltpu.VMEM((8, 128), jnp.int32),
            pltpu.VMEM((8, 128), jnp.int32),
        ],
        compiler_params=pltpu.CompilerParams(
            dimension_semantics=("arbitrary", "arbitrary"),
        ),
    )(logits)
    return out[0, 0]


# speedup vs baseline: 406.0437x; 1.0003x over previous
import functools

import jax
import jax.numpy as jnp
from jax.experimental import pallas as pl
from jax.experimental.pallas import tpu as pltpu


def _loss_body(lref, oref, nacc, sacc, carry, *, nc, rows, w, h, nchunk):
    ncls = nc - 1
    pack = 2048
    b = pl.program_id(0)
    j = pl.program_id(1)

    x = lref[0]

    best = x[0]
    idx = jnp.zeros((rows, w), jnp.int32)
    for c in range(1, nc):
        v = x[c]
        m = v > best
        best = jnp.where(m, v, best)
        idx = jnp.where(m, c, idx)

    colp = jax.lax.broadcasted_iota(
        jnp.int32, (rows, w), 1).astype(jnp.float32) + 1.0
    colr = jnp.float32(w + 1) - colp
    ns_l, mn_l, mx_l = [], [], []
    for c in range(1, nc):
        mf = jnp.where(idx == c, 1.0, 0.0)
        ns_l.append(jnp.sum(mf, axis=1, keepdims=True))
        mx_l.append(jnp.max(mf * colp, axis=1, keepdims=True))
        mn_l.append(jnp.max(mf * colr, axis=1, keepdims=True))
    ns = jnp.concatenate(ns_l, axis=1).astype(jnp.int32)
    mn = w - jnp.concatenate(mn_l, axis=1).astype(jnp.int32)
    mx = jnp.concatenate(mx_l, axis=1).astype(jnp.int32) - 1

    grow = j * rows + jax.lax.broadcasted_iota(jnp.int32, (rows, ncls), 0)
    occ = ns > 0
    first = mn - grow
    last = mx - grow
    srow = jnp.where(occ, mx - mn, 0)
    packed = jnp.where(occ, (grow + 1) * pack + (last + h), -1)

    @pl.when(j == 0)
    def _():
        nacc[:1, :ncls] = jnp.zeros((1, ncls), jnp.int32)
        sacc[:1, :ncls] = jnp.zeros((1, ncls), jnp.int32)
        carry[:1, :ncls] = jnp.full((1, ncls), -1, jnp.int32)

    e = jnp.concatenate([carry[:1, :ncls], packed[:-1]], axis=0)
    k = 1
    while k < rows:
        pad = jnp.full((k, ncls), -1, jnp.int32)
        e = jnp.maximum(e, jnp.concatenate([pad, e[:-k]], axis=0))
        k *= 2
    prev_ok = e >= 0
    prev_last = (e & (pack - 1)) - h
    cross = jnp.where(occ & prev_ok, jnp.abs(first - prev_last), 0)

    nacc[:1, :ncls] = nacc[:1, :ncls] + jnp.sum(ns, axis=0, keepdims=True)
    sacc[:1, :ncls] = sacc[:1, :ncls] + jnp.sum(srow + cross, axis=0,
                                                keepdims=True)
    carry[:1, :ncls] = jnp.maximum(e[-1:, :], packed[-1:, :])

    @pl.when(jnp.logical_and(b == 0, j == 0))
    def _():
        oref[:, :] = jnp.zeros((1, 1), jnp.float32)

    @pl.when(j == nchunk - 1)
    def _():
        nf = nacc[:1, :ncls].astype(jnp.float32)
        sf = sacc[:1, :ncls].astype(jnp.float32)
        mean = sf / jnp.maximum(nf - 1.0, 1.0)
        contrib = jnp.where(nf >= 2.0, mean / (nf + 1.0), 0.0)
        oref[:, :] = oref[:, :] + jnp.sum(contrib, axis=1, keepdims=True)


def kernel(logits, labels):
    del labels
    bs, nc, h, w = logits.shape
    rows = 256
    nchunk = h // rows

    body = functools.partial(_loss_body, nc=nc, rows=rows, w=w, h=h,
                             nchunk=nchunk)
    out = pl.pallas_call(
        body,
        grid=(bs, nchunk),
        in_specs=[
            pl.BlockSpec((1, nc, rows, w), lambda b, j: (b, 0, j, 0)),
        ],
        out_specs=pl.BlockSpec((1, 1), lambda b, j: (0, 0)),
        out_shape=jax.ShapeDtypeStruct((1, 1), jnp.float32),
        scratch_shapes=[
            pltpu.VMEM((8, 128), jnp.int32),
            pltpu.VMEM((8, 128), jnp.int32),
            pltpu.VMEM((8, 128), jnp.int32),
        ],
        compiler_params=pltpu.CompilerParams(
            dimension_semantics=("arbitrary", "arbitrary"),
        ),
    )(logits)
    return out[0, 0]
